# baseline (speedup 1.0000x reference)
import functools
import math

import jax
import jax.numpy as jnp
from jax import lax
from jax.experimental import pallas as pl
from jax.experimental.pallas import tpu as pltpu
from jax.experimental.pallas import tpu_sc as plsc

_D = 64
_SCALE = 8.0
_LANES = 16
_CHUNK = 400
_CPS = 32
_TC_BLOCK = 64


@functools.cache
def _make_sc_gather(B, V, D, chunk, cps):
    NC, NS = 2, 16
    NW = NC * NS
    b_per_w = B // NW
    n_chunks = b_per_w // chunk
    n_super = n_chunks // cps
    assert b_per_w * NW == B and n_super * cps == n_chunks
    assert n_super % 2 == 0 and cps % 2 == 0 and chunk % 2 == 0
    hchunk = chunk // 2
    sup_len = cps * chunk
    mesh = plsc.VectorSubcoreMesh(core_axis_name="c", subcore_axis_name="s")

    @functools.partial(
        pl.kernel,
        out_type=jax.ShapeDtypeStruct((B // 2, 2 * D), jnp.float32),
        mesh=mesh,
        scratch_types=[
            pltpu.VMEM((sup_len,), jnp.int32),
            pltpu.VMEM((sup_len,), jnp.int32),
            pltpu.VMEM((chunk, D), jnp.float32),
            pltpu.VMEM((chunk, D), jnp.float32),
            pltpu.VMEM((hchunk, 2 * D), jnp.float32),
            pltpu.VMEM((hchunk, 2 * D), jnp.float32),
            pltpu.SemaphoreType.DMA,
            pltpu.SemaphoreType.DMA,
            pltpu.SemaphoreType.DMA,
            pltpu.SemaphoreType.DMA,
            pltpu.SemaphoreType.DMA,
            pltpu.SemaphoreType.DMA,
        ],
        compiler_params=pltpu.CompilerParams(use_tc_tiling_on_sc=False),
    )
    def sc_gather(x_hbm, table_hbm, out_hbm, idxb0, idxb1, big0, big1,
                  comp0, comp1, si0, si1, sg0, sg1, ss0, ss1):
        wid = lax.axis_index("s") * NC + lax.axis_index("c")
        base = wid * b_per_w
        idxb = (idxb0, idxb1)
        sidx = (si0, si1)
        bigs = (big0, big1)
        comps = (comp0, comp1)
        sg = (sg0, sg1)
        ss = (ss0, ss1)

        def idx_load(s, ib):
            pltpu.async_copy(
                x_hbm.at[pl.ds(base + s * sup_len, sup_len)], idxb[ib],
                sidx[ib])

        def idx_wait(ib):
            pltpu.make_async_copy(
                x_hbm.at[pl.ds(base, sup_len)], idxb[ib], sidx[ib]).wait()

        def gather_start(c, ib, bb):
            pltpu.async_copy(
                table_hbm.at[idxb[ib].at[pl.ds(c * chunk, chunk)]],
                bigs[bb], sg[bb])

        def gather_wait(bb):
            pltpu.make_async_copy(
                table_hbm.at[idxb[0].at[pl.ds(0, chunk)]], bigs[bb],
                sg[bb]).wait()

        def pack(bb):
            big, comp = bigs[bb], comps[bb]

            def pair_body(p, _):
                for h in range(2):
                    for j in range(D // _LANES):
                        src = pl.ds(j * _LANES, _LANES)
                        dst = pl.ds(h * D + j * _LANES, _LANES)
                        comp[p, dst] = big[2 * p + h, src] * _SCALE
                return ()

            lax.fori_loop(0, hchunk, pair_body, (), unroll=8)

        def scatter_start(s, c, bb):
            ph0 = (base + s * sup_len + c * chunk) // 2
            pltpu.async_copy(comps[bb], out_hbm.at[pl.ds(ph0, hchunk)],
                             ss[bb])

        def scatter_wait(bb):
            pltpu.make_async_copy(
                comps[bb], out_hbm.at[pl.ds(0, hchunk)], ss[bb]).wait()

        idx_load(0, 0)
        idx_load(1, 1)

        def super_pair(s2, _):
            for sb in range(2):
                s = 2 * s2 + sb
                idx_wait(sb)
                gather_start(0, sb, 0)

                def chunk_pair(c2, _):
                    for cb in range(2):
                        c = 2 * c2 + cb

                        @pl.when(c < cps - 1)
                        def _():
                            gather_start(c + 1, sb, 1 - cb)

                        gather_wait(cb)
                        if sb == 0:
                            @pl.when((s2 > 0) | (c2 > 0))
                            def _():
                                scatter_wait(cb)
                        else:
                            scatter_wait(cb)
                        pack(cb)
                        scatter_start(s, c, cb)
                    return ()

                lax.fori_loop(0, cps // 2, chunk_pair, ())

                @pl.when(s + 2 < n_super)
                def _():
                    idx_load(s + 2, sb)
            return ()

        lax.fori_loop(0, n_super // 2, super_pair, ())
        scatter_wait(0)
        scatter_wait(1)

    return sc_gather


@functools.cache
def _make_tc_relayout(S0, S1, D, G):
    B = S0 * S1
    rows_in = G * S1 * D // (2 * _D)

    def body(i_ref, o_ref):
        x = i_ref[...]
        a = x[:, None, :D]
        b = x[:, None, D:]
        o_ref[...] = jnp.concatenate([a, b], axis=1).reshape(G, S1, D)

    return pl.pallas_call(
        body,
        grid=(S0 // G,),
        in_specs=[pl.BlockSpec((rows_in, 2 * _D), lambda i: (i, 0))],
        out_specs=pl.BlockSpec((G, S1, D), lambda i: (i, 0, 0)),
        out_shape=jax.ShapeDtypeStruct((S0, S1, D), jnp.float32),
    )


def kernel(x, table):
    S0, S1 = x.shape
    V, D = table.shape
    B = S0 * S1
    flat = x.reshape(B).astype(jnp.int32)
    y2 = _make_sc_gather(B, V, D, _CHUNK, _CPS)(flat, table)
    return _make_tc_relayout(S0, S1, D, _TC_BLOCK)(y2)
